# Initial kernel scaffold; baseline (speedup 1.0000x reference)
import jax
import jax.numpy as jnp
from jax.experimental import pallas as pl


def kernel(x, e1_wtop, e1_wdiff, e1_b, e1_w2, e1_b2, e2_wtop, e2_wdiff, e2_b, e2_w2, e2_b2, e3_wtop, e3_wdiff, e3_b, w6, b6, w8g, w8p, b8, w9, b9, w10, b10, w11p):
    raise NotImplementedError("write your pallas kernel here")


# speedup vs baseline: 2.2567x; 2.2567x over previous
import jax
import jax.numpy as jnp
from jax.experimental import pallas as pl
from jax.experimental.pallas import tpu as pltpu

_VMEM_LIMIT = 48 * 1024 * 1024
_BF = jnp.bfloat16


def _lrelu(h):
    return jnp.maximum(h, 0.2 * h)


def _knn_idx(xt, k):
    inner = jnp.matmul(xt, jnp.swapaxes(xt, 1, 2))
    sq = jnp.sum(xt * xt, axis=-1, keepdims=True)
    neg_dist = 2.0 * inner - sq - jnp.swapaxes(sq, 1, 2)
    return jax.lax.top_k(neg_dist, k)[1]


def _vmem_gather(tbl, idx_row):
    C = tbl.shape[0]
    n_chunks = max(1, idx_row.shape[1] // 128)
    outs = []
    for o in range(n_chunks):
        io = idx_row[:, o * 128:(o + 1) * 128]
        lane = jnp.broadcast_to(io & 127, (C, 128))
        hi = jnp.broadcast_to(io >> 7, (C, 128))
        g = jnp.take_along_axis(tbl[:, 0:128], lane, axis=1)
        for c in range(1, n_chunks):
            gc = jnp.take_along_axis(tbl[:, c * 128:(c + 1) * 128], lane,
                                     axis=1)
            g = jnp.where(hi == c, gc, g)
        outs.append(g)
    if n_chunks == 1:
        return outs[0]
    return jnp.concatenate(outs, axis=1)


def _edge2_kernel(tbl_ref, idx_ref, cen_ref, w2_ref, b2_ref, out_ref):
    C1, N = tbl_ref.shape[1], tbl_ref.shape[2]
    K = idx_ref.shape[1]
    C2 = w2_ref.shape[1]
    tbl, idx, cen = tbl_ref[0], idx_ref[0], cen_ref[0]
    parts = []
    for j in range(K):
        g = _vmem_gather(tbl, idx[j:j + 1, :])
        parts.append(jnp.transpose(_lrelu(g + cen)))
    h1 = jnp.concatenate(parts, axis=0)
    h2 = jnp.dot(h1, w2_ref[...],
                 preferred_element_type=jnp.float32)
    m = jnp.max(h2.reshape(K, N, C2), axis=0)
    out_ref[0] = _lrelu(m + b2_ref[...])


def _edge1_kernel(tbl_ref, idx_ref, cen_ref, out_ref):
    C1, N = tbl_ref.shape[1], tbl_ref.shape[2]
    K = idx_ref.shape[1]
    tbl, idx = tbl_ref[0], idx_ref[0]
    m = _vmem_gather(tbl, idx[0:1, :])
    for j in range(1, K):
        m = jnp.maximum(m, _vmem_gather(tbl, idx[j:j + 1, :]))
    out_ref[0] = jnp.transpose(_lrelu(m + cen_ref[0]))


def _edge_conv(projT, idx_t, cenT, w2, b2):
    B, C1, N = projT.shape
    K = idx_t.shape[1]
    if w2 is None:
        kfn, cout, extra, especs = _edge1_kernel, C1, [], []
    else:
        kfn, cout = _edge2_kernel, w2.shape[1]
        extra = [w2, b2]
        especs = [pl.BlockSpec(w2.shape, lambda b: (0, 0)),
                  pl.BlockSpec(b2.shape, lambda b: (0, 0))]
    return pl.pallas_call(
        kfn,
        out_shape=jax.ShapeDtypeStruct((B, N, cout), jnp.float32),
        grid=(B,),
        in_specs=[pl.BlockSpec((1, C1, N), lambda b: (b, 0, 0)),
                  pl.BlockSpec((1, K, N), lambda b: (b, 0, 0)),
                  pl.BlockSpec((1, C1, N), lambda b: (b, 0, 0)),
                  *especs],
        out_specs=pl.BlockSpec((1, N, cout), lambda b: (b, 0, 0)),
        compiler_params=pltpu.CompilerParams(
            dimension_semantics=("parallel",),
            vmem_limit_bytes=_VMEM_LIMIT),
    )(projT, idx_t, cenT, *extra)


def _head_kernel(x1_ref, x2_ref, x3_ref,
                 w6a_ref, w6b_ref, w6c_ref, b6_ref,
                 w8g_ref, b8_ref, w8a_ref, w8b_ref, w8c_ref,
                 w9_ref, b9_ref, w10_ref, b10_ref, w11_ref,
                 out_ref):
    x1, x2, x3 = x1_ref[0], x2_ref[0], x3_ref[0]
    f32 = jnp.float32
    h6 = (jnp.dot(x1, w6a_ref[...], preferred_element_type=f32)
          + jnp.dot(x2, w6b_ref[...], preferred_element_type=f32)
          + jnp.dot(x3, w6c_ref[...], preferred_element_type=f32))
    g = _lrelu(jnp.max(h6, axis=0, keepdims=True) + b6_ref[...])
    gb8 = jnp.dot(g, w8g_ref[...], preferred_element_type=f32) + b8_ref[...]
    h8 = (jnp.dot(x1, w8a_ref[...], preferred_element_type=f32)
          + jnp.dot(x2, w8b_ref[...], preferred_element_type=f32)
          + jnp.dot(x3, w8c_ref[...], preferred_element_type=f32))
    h8 = _lrelu(h8 + gb8)
    h9 = _lrelu(jnp.dot(h8.astype(_BF), w9_ref[...],
                        preferred_element_type=f32) + b9_ref[...])
    h10 = _lrelu(jnp.dot(h9.astype(_BF), w10_ref[...],
                         preferred_element_type=f32) + b10_ref[...])
    h11 = jnp.dot(h10.astype(_BF), w11_ref[...],
                  preferred_element_type=f32)
    out_ref[0] = jnp.transpose(h11)[:out_ref.shape[1], :]


def _head(x1b, x2b, x3b, w6, b6, w8g, b8, w8p, w9, b9, w10, b10, w11p,
          num_part):
    B, N, _ = x1b.shape
    w6a, w6b, w6c = (w6[:64].astype(_BF), w6[64:128].astype(_BF),
                     w6[128:].astype(_BF))
    w8a, w8b, w8c = (w8p[:64].astype(_BF), w8p[64:128].astype(_BF),
                     w8p[128:].astype(_BF))
    weights = [w6a, w6b, w6c, b6, w8g, b8, w8a, w8b, w8c,
               w9.astype(_BF), b9, w10.astype(_BF), b10, w11p.astype(_BF)]
    return pl.pallas_call(
        _head_kernel,
        out_shape=jax.ShapeDtypeStruct((B, num_part, N), jnp.float32),
        grid=(B,),
        in_specs=[pl.BlockSpec((1, N, 64), lambda b: (b, 0, 0))] * 3
                 + [pl.BlockSpec(w.shape, lambda b: (0,) * w.ndim)
                    for w in weights],
        out_specs=pl.BlockSpec((1, num_part, N), lambda b: (b, 0, 0)),
        compiler_params=pltpu.CompilerParams(
            dimension_semantics=("parallel",),
            vmem_limit_bytes=_VMEM_LIMIT),
    )(x1b, x2b, x3b, *weights)


@jax.jit
def kernel(x, e1_wtop, e1_wdiff, e1_b, e1_w2, e1_b2,
           e2_wtop, e2_wdiff, e2_b, e2_w2, e2_b2,
           e3_wtop, e3_wdiff, e3_b,
           w6, b6, w8g, w8p, b8, w9, b9, w10, b10, w11p):
    k, num_part = 40, 50
    xt = jnp.transpose(x.astype(jnp.float32), (0, 2, 1))

    def block(feat, wtop, wdiff, bias, w2, b2):
        idx_t = jnp.transpose(_knn_idx(feat, k), (0, 2, 1))
        projT = jnp.einsum("bnc,cd->bdn", feat, wtop)
        cenT = (jnp.einsum("bnc,cd->bdn", feat, wdiff)
                + bias.T[None, :, :])
        return _edge_conv(projT, idx_t, cenT, w2, b2)

    x1 = block(xt, e1_wtop, e1_wdiff, e1_b, e1_w2, e1_b2)
    x2 = block(x1, e2_wtop, e2_wdiff, e2_b, e2_w2, e2_b2)
    x3 = block(x2, e3_wtop, e3_wdiff, e3_b, None, None)

    return _head(x1.astype(_BF), x2.astype(_BF), x3.astype(_BF),
                 w6, b6, w8g, b8, w8p, w9, b9, w10, b10, w11p, num_part)
